# baseline (speedup 1.0000x reference)
import functools

import jax
import jax.numpy as jnp
from jax import lax
from jax.experimental import pallas as pl
from jax.experimental.pallas import tpu as pltpu
from jax.experimental.pallas import tpu_sc as plsc

_LANES = 16

_K1 = 0
_K2 = 123
_K3 = _K1 ^ _K2 ^ 0x1BD11BDA
_ROT0 = (13, 15, 26, 6)
_ROT1 = (17, 29, 16, 24)


def _rotl(x, r):
    return (x << jnp.uint32(r)) | (x >> jnp.uint32(32 - r))


def _threefry2x32(x0, x1):
    ks = (jnp.uint32(_K1), jnp.uint32(_K2), jnp.uint32(_K3))
    x0 = x0 + ks[0]
    x1 = x1 + ks[1]

    def rounds(x0, x1, rots):
        for r in rots:
            x0 = x0 + x1
            x1 = x0 ^ _rotl(x1, r)
        return x0, x1

    x0, x1 = rounds(x0, x1, _ROT0)
    x0, x1 = x0 + ks[1], x1 + ks[2] + jnp.uint32(1)
    x0, x1 = rounds(x0, x1, _ROT1)
    x0, x1 = x0 + ks[2], x1 + ks[0] + jnp.uint32(2)
    x0, x1 = rounds(x0, x1, _ROT0)
    x0, x1 = x0 + ks[0], x1 + ks[1] + jnp.uint32(3)
    x0, x1 = rounds(x0, x1, _ROT1)
    x0, x1 = x0 + ks[1], x1 + ks[2] + jnp.uint32(4)
    x0, x1 = rounds(x0, x1, _ROT0)
    x0, x1 = x0 + ks[2], x1 + ks[0] + jnp.uint32(5)
    return x0, x1


def _sc_body(num_cores, rows_per_worker, num_actions,
             theta_hbm, logits_hbm, baseline_hbm, action_hbm,
             theta_v, logits_v, baseline_v, action_v):
    wid = lax.axis_index("s") * num_cores + lax.axis_index("c")
    row_base = wid * rows_per_worker

    pltpu.sync_copy(theta_hbm, theta_v)
    zeros_f = (theta_v[pl.ds(0, _LANES)] + theta_v[pl.ds(_LANES, _LANES)]) * 0.0

    lane = lax.iota(jnp.int32, _LANES)
    rows_u32 = (lane + jnp.int32(row_base)).astype(jnp.uint32)

    best_bits = jnp.zeros((_LANES,), jnp.uint32)
    best_act = jnp.zeros((_LANES,), jnp.int32)
    zero_u32 = jnp.zeros((_LANES,), jnp.uint32)
    for a in range(num_actions):
        cnt = rows_u32 * jnp.uint32(num_actions) + jnp.uint32(a)
        b0, b1 = _threefry2x32(zero_u32, cnt)
        key23 = (b0 ^ b1) >> jnp.uint32(9)
        take = key23 > best_bits
        best_bits = jnp.where(take, key23, best_bits)
        best_act = jnp.where(take, jnp.full((_LANES,), a, jnp.int32), best_act)

    action_v[...] = best_act
    baseline_v[...] = zeros_f
    for c in range(num_actions):
        logits_v[pl.ds(c * _LANES, _LANES)] = zeros_f

    n_logits = rows_per_worker * num_actions
    pltpu.sync_copy(logits_v, logits_hbm.at[pl.ds(row_base * num_actions, n_logits)])
    pltpu.sync_copy(baseline_v, baseline_hbm.at[pl.ds(row_base, rows_per_worker)])
    pltpu.sync_copy(action_v, action_hbm.at[pl.ds(row_base, rows_per_worker)])


def kernel(observation, theta, core_state):
    T, B = observation.shape[0], observation.shape[1]
    A = theta.shape[0]
    n_rows = T * B

    info = plsc.get_sparse_core_info()
    num_cores, num_subcores = info.num_cores, info.num_subcores
    num_workers = num_cores * num_subcores
    assert n_rows % num_workers == 0
    rows_per_worker = n_rows // num_workers
    assert rows_per_worker == _LANES

    theta_pad = jnp.zeros((2 * _LANES,), jnp.float32).at[:A].set(theta)

    mesh = plsc.VectorSubcoreMesh(core_axis_name="c", subcore_axis_name="s")
    run = pl.kernel(
        functools.partial(_sc_body, num_cores, rows_per_worker, A),
        out_type=(
            jax.ShapeDtypeStruct((n_rows * A,), jnp.float32),
            jax.ShapeDtypeStruct((n_rows,), jnp.float32),
            jax.ShapeDtypeStruct((n_rows,), jnp.int32),
        ),
        mesh=mesh,
        scratch_types=(
            pltpu.VMEM((2 * _LANES,), jnp.float32),
            pltpu.VMEM((rows_per_worker * A,), jnp.float32),
            pltpu.VMEM((rows_per_worker,), jnp.float32),
            pltpu.VMEM((rows_per_worker,), jnp.int32),
        ),
    )
    logits_flat, baseline_flat, action_flat = run(theta_pad)
    policy_logits = logits_flat.reshape(T, B, A)
    baseline = baseline_flat.reshape(T, B)
    action = action_flat.reshape(T, B)
    return (policy_logits, baseline, action)


# speedup vs baseline: 1.0961x; 1.0961x over previous
import functools

import jax
import jax.numpy as jnp
from jax import lax
from jax.experimental import pallas as pl
from jax.experimental.pallas import tpu as pltpu
from jax.experimental.pallas import tpu_sc as plsc

_LANES = 16

_K1 = 0
_K2 = 123
_K3 = _K1 ^ _K2 ^ 0x1BD11BDA
_ROT0 = (13, 15, 26, 6)
_ROT1 = (17, 29, 16, 24)


def _rotl(x, r):
    return (x << jnp.uint32(r)) | (x >> jnp.uint32(32 - r))


def _threefry2x32(x0, x1):
    ks = (jnp.uint32(_K1), jnp.uint32(_K2), jnp.uint32(_K3))
    x0 = x0 + ks[0]
    x1 = x1 + ks[1]

    def rounds(x0, x1, rots):
        for r in rots:
            x0 = x0 + x1
            x1 = x0 ^ _rotl(x1, r)
        return x0, x1

    x0, x1 = rounds(x0, x1, _ROT0)
    x0, x1 = x0 + ks[1], x1 + ks[2] + jnp.uint32(1)
    x0, x1 = rounds(x0, x1, _ROT1)
    x0, x1 = x0 + ks[2], x1 + ks[0] + jnp.uint32(2)
    x0, x1 = rounds(x0, x1, _ROT0)
    x0, x1 = x0 + ks[0], x1 + ks[1] + jnp.uint32(3)
    x0, x1 = rounds(x0, x1, _ROT1)
    x0, x1 = x0 + ks[1], x1 + ks[2] + jnp.uint32(4)
    x0, x1 = rounds(x0, x1, _ROT0)
    x0, x1 = x0 + ks[2], x1 + ks[0] + jnp.uint32(5)
    return x0, x1


def _sc_body(num_cores, rows_per_worker, num_actions,
             logits_hbm, baseline_hbm, action_hbm,
             logits_v, baseline_v, action_v, sem):
    wid = lax.axis_index("s") * num_cores + lax.axis_index("c")
    row_base = wid * rows_per_worker

    lane = lax.iota(jnp.int32, _LANES)
    rows_u32 = (lane + jnp.int32(row_base)).astype(jnp.uint32)
    zeros_f = lane.astype(jnp.float32) * 0.0

    zero_u32 = jnp.zeros((_LANES,), jnp.uint32)

    def step(a, carry):
        best_bits, best_act = carry
        cnt = rows_u32 * jnp.uint32(num_actions) + a.astype(jnp.uint32)
        b0, b1 = _threefry2x32(zero_u32, cnt)
        key23 = (b0 ^ b1) >> jnp.uint32(9)
        take = key23 > best_bits
        best_bits = jnp.where(take, key23, best_bits)
        best_act = jnp.where(take, jnp.broadcast_to(a, (_LANES,)), best_act)
        return best_bits, best_act

    _, best_act = lax.fori_loop(
        0, num_actions, step,
        (jnp.zeros((_LANES,), jnp.uint32), jnp.zeros((_LANES,), jnp.int32)))

    action_v[...] = best_act
    baseline_v[...] = zeros_f

    def fill(c, _):
        logits_v[pl.ds(c * _LANES, _LANES)] = zeros_f
        return 0

    lax.fori_loop(0, num_actions, fill, 0)

    n_logits = rows_per_worker * num_actions
    cp1 = pltpu.async_copy(
        logits_v, logits_hbm.at[pl.ds(row_base * num_actions, n_logits)], sem)
    cp2 = pltpu.async_copy(
        baseline_v, baseline_hbm.at[pl.ds(row_base, rows_per_worker)], sem)
    cp3 = pltpu.async_copy(
        action_v, action_hbm.at[pl.ds(row_base, rows_per_worker)], sem)
    cp1.wait()
    cp2.wait()
    cp3.wait()


def kernel(observation, theta, core_state):
    T, B = observation.shape[0], observation.shape[1]
    A = theta.shape[0]
    n_rows = T * B

    info = plsc.get_sparse_core_info()
    num_cores, num_subcores = info.num_cores, info.num_subcores
    num_workers = num_cores * num_subcores
    assert n_rows % num_workers == 0
    rows_per_worker = n_rows // num_workers
    assert rows_per_worker == _LANES

    mesh = plsc.VectorSubcoreMesh(core_axis_name="c", subcore_axis_name="s")
    run = pl.kernel(
        functools.partial(_sc_body, num_cores, rows_per_worker, A),
        out_type=(
            jax.ShapeDtypeStruct((n_rows * A,), jnp.float32),
            jax.ShapeDtypeStruct((n_rows,), jnp.float32),
            jax.ShapeDtypeStruct((n_rows,), jnp.int32),
        ),
        mesh=mesh,
        scratch_types=(
            pltpu.VMEM((rows_per_worker * A,), jnp.float32),
            pltpu.VMEM((rows_per_worker,), jnp.float32),
            pltpu.VMEM((rows_per_worker,), jnp.int32),
            pltpu.SemaphoreType.DMA,
        ),
    )
    logits_flat, baseline_flat, action_flat = run()
    policy_logits = logits_flat.reshape(T, B, A)
    baseline = baseline_flat.reshape(T, B)
    action = action_flat.reshape(T, B)
    return (policy_logits, baseline, action)


# speedup vs baseline: 1.1438x; 1.0435x over previous
import functools

import jax
import jax.numpy as jnp
from jax import lax
from jax.experimental import pallas as pl
from jax.experimental.pallas import tpu as pltpu
from jax.experimental.pallas import tpu_sc as plsc

_LANES = 16

_K1 = 0
_K2 = 123
_K3 = _K1 ^ _K2 ^ 0x1BD11BDA
_ROT0 = (13, 15, 26, 6)
_ROT1 = (17, 29, 16, 24)


def _rotl(x, r):
    return (x << jnp.uint32(r)) | (x >> jnp.uint32(32 - r))


def _threefry2x32(x0, x1):
    ks = (jnp.uint32(_K1), jnp.uint32(_K2), jnp.uint32(_K3))
    x0 = x0 + ks[0]
    x1 = x1 + ks[1]

    def rounds(x0, x1, rots):
        for r in rots:
            x0 = x0 + x1
            x1 = x0 ^ _rotl(x1, r)
        return x0, x1

    x0, x1 = rounds(x0, x1, _ROT0)
    x0, x1 = x0 + ks[1], x1 + ks[2] + jnp.uint32(1)
    x0, x1 = rounds(x0, x1, _ROT1)
    x0, x1 = x0 + ks[2], x1 + ks[0] + jnp.uint32(2)
    x0, x1 = rounds(x0, x1, _ROT0)
    x0, x1 = x0 + ks[0], x1 + ks[1] + jnp.uint32(3)
    x0, x1 = rounds(x0, x1, _ROT1)
    x0, x1 = x0 + ks[1], x1 + ks[2] + jnp.uint32(4)
    x0, x1 = rounds(x0, x1, _ROT0)
    x0, x1 = x0 + ks[2], x1 + ks[0] + jnp.uint32(5)
    return x0, x1


def _sc_body(num_cores, rows_per_worker, num_actions,
             logits_hbm, baseline_hbm, action_hbm,
             logits_v, baseline_v, action_v, sem):
    wid = lax.axis_index("s") * num_cores + lax.axis_index("c")
    row_base = wid * rows_per_worker

    lane = lax.iota(jnp.int32, _LANES)
    zeros_f = lane.astype(jnp.float32) * 0.0
    zero_u32 = jnp.zeros((_LANES,), jnp.uint32)

    for g in range(rows_per_worker // _LANES):
        rows_u32 = (lane + jnp.int32(row_base + g * _LANES)).astype(jnp.uint32)

        def step(a, carry):
            best_bits, best_act = carry
            cnt = rows_u32 * jnp.uint32(num_actions) + a.astype(jnp.uint32)
            b0, b1 = _threefry2x32(zero_u32, cnt)
            key23 = (b0 ^ b1) >> jnp.uint32(9)
            take = key23 > best_bits
            best_bits = jnp.where(take, key23, best_bits)
            best_act = jnp.where(take, jnp.broadcast_to(a, (_LANES,)), best_act)
            return best_bits, best_act

        _, best_act = lax.fori_loop(
            0, num_actions, step,
            (jnp.zeros((_LANES,), jnp.uint32), jnp.zeros((_LANES,), jnp.int32)))
        action_v[pl.ds(g * _LANES, _LANES)] = best_act
        baseline_v[pl.ds(g * _LANES, _LANES)] = zeros_f

    def fill(c, _):
        logits_v[pl.ds(c * _LANES, _LANES)] = zeros_f
        return 0

    lax.fori_loop(0, (rows_per_worker * num_actions) // _LANES, fill, 0)

    n_logits = rows_per_worker * num_actions
    cp1 = pltpu.async_copy(
        logits_v, logits_hbm.at[pl.ds(row_base * num_actions, n_logits)], sem)
    cp2 = pltpu.async_copy(
        baseline_v, baseline_hbm.at[pl.ds(row_base, rows_per_worker)], sem)
    cp3 = pltpu.async_copy(
        action_v, action_hbm.at[pl.ds(row_base, rows_per_worker)], sem)
    cp1.wait()
    cp2.wait()
    cp3.wait()


def kernel(observation, theta, core_state):
    T, B = observation.shape[0], observation.shape[1]
    A = theta.shape[0]
    n_rows = T * B

    info = plsc.get_sparse_core_info()
    num_cores, num_subcores = 1, info.num_subcores
    num_workers = num_cores * num_subcores
    assert n_rows % (num_workers * _LANES) == 0
    rows_per_worker = n_rows // num_workers

    mesh = plsc.VectorSubcoreMesh(
        core_axis_name="c", subcore_axis_name="s", num_cores=num_cores)
    run = pl.kernel(
        functools.partial(_sc_body, num_cores, rows_per_worker, A),
        out_type=(
            jax.ShapeDtypeStruct((n_rows * A,), jnp.float32),
            jax.ShapeDtypeStruct((n_rows,), jnp.float32),
            jax.ShapeDtypeStruct((n_rows,), jnp.int32),
        ),
        mesh=mesh,
        scratch_types=(
            pltpu.VMEM((rows_per_worker * A,), jnp.float32),
            pltpu.VMEM((rows_per_worker,), jnp.float32),
            pltpu.VMEM((rows_per_worker,), jnp.int32),
            pltpu.SemaphoreType.DMA,
        ),
    )
    logits_flat, baseline_flat, action_flat = run()
    policy_logits = logits_flat.reshape(T, B, A)
    baseline = baseline_flat.reshape(T, B)
    action = action_flat.reshape(T, B)
    return (policy_logits, baseline, action)


# speedup vs baseline: 1.2713x; 1.1114x over previous
import functools

import jax
import jax.numpy as jnp
from jax import lax
from jax.experimental import pallas as pl
from jax.experimental.pallas import tpu as pltpu
from jax.experimental.pallas import tpu_sc as plsc

_LANES = 16

_K1 = 0
_K2 = 123
_K3 = _K1 ^ _K2 ^ 0x1BD11BDA
_ROT0 = (13, 15, 26, 6)
_ROT1 = (17, 29, 16, 24)


def _rotl(x, r):
    return (x << jnp.uint32(r)) | (x >> jnp.uint32(32 - r))


def _threefry2x32(x0, x1):
    ks = (jnp.uint32(_K1), jnp.uint32(_K2), jnp.uint32(_K3))
    x0 = x0 + ks[0]
    x1 = x1 + ks[1]

    def rounds(x0, x1, rots):
        for r in rots:
            x0 = x0 + x1
            x1 = x0 ^ _rotl(x1, r)
        return x0, x1

    x0, x1 = rounds(x0, x1, _ROT0)
    x0, x1 = x0 + ks[1], x1 + ks[2] + jnp.uint32(1)
    x0, x1 = rounds(x0, x1, _ROT1)
    x0, x1 = x0 + ks[2], x1 + ks[0] + jnp.uint32(2)
    x0, x1 = rounds(x0, x1, _ROT0)
    x0, x1 = x0 + ks[0], x1 + ks[1] + jnp.uint32(3)
    x0, x1 = rounds(x0, x1, _ROT1)
    x0, x1 = x0 + ks[1], x1 + ks[2] + jnp.uint32(4)
    x0, x1 = rounds(x0, x1, _ROT0)
    x0, x1 = x0 + ks[2], x1 + ks[0] + jnp.uint32(5)
    return x0, x1


def _sc_sampler(num_cores, rows_per_worker, num_actions,
                action_hbm, action_v):
    wid = lax.axis_index("s") * num_cores + lax.axis_index("c")
    row_base = wid * rows_per_worker

    lane = lax.iota(jnp.int32, _LANES)
    zero_u32 = jnp.zeros((_LANES,), jnp.uint32)

    for g in range(rows_per_worker // _LANES):
        rows_u32 = (lane + jnp.int32(row_base + g * _LANES)).astype(jnp.uint32)

        def step(a, carry):
            best_bits, best_act = carry
            cnt = rows_u32 * jnp.uint32(num_actions) + a.astype(jnp.uint32)
            b0, b1 = _threefry2x32(zero_u32, cnt)
            key23 = (b0 ^ b1) >> jnp.uint32(9)
            take = key23 > best_bits
            best_bits = jnp.where(take, key23, best_bits)
            best_act = jnp.where(take, jnp.broadcast_to(a, (_LANES,)), best_act)
            return best_bits, best_act

        _, best_act = lax.fori_loop(
            0, num_actions, step,
            (jnp.zeros((_LANES,), jnp.uint32), jnp.zeros((_LANES,), jnp.int32)))
        action_v[pl.ds(g * _LANES, _LANES)] = best_act

    pltpu.sync_copy(action_v, action_hbm.at[pl.ds(row_base, rows_per_worker)])


def _tc_dense(theta_ref, logits_ref, baseline_ref):
    z = theta_ref[...] * 0.0
    t, b, a = logits_ref.shape
    logits_ref[...] = jnp.broadcast_to(z[None, None, :], (t, b, a))
    baseline_ref[...] = jnp.broadcast_to(jnp.sum(z), (t, b))


def kernel(observation, theta, core_state):
    T, B = observation.shape[0], observation.shape[1]
    A = theta.shape[0]
    n_rows = T * B

    info = plsc.get_sparse_core_info()
    num_cores, num_subcores = 1, info.num_subcores
    num_workers = num_cores * num_subcores
    assert n_rows % (num_workers * _LANES) == 0
    rows_per_worker = n_rows // num_workers

    mesh = plsc.VectorSubcoreMesh(
        core_axis_name="c", subcore_axis_name="s", num_cores=num_cores)
    sample = pl.kernel(
        functools.partial(_sc_sampler, num_cores, rows_per_worker, A),
        out_type=jax.ShapeDtypeStruct((n_rows,), jnp.int32),
        mesh=mesh,
        scratch_types=(pltpu.VMEM((rows_per_worker,), jnp.int32),),
    )

    dense = pl.pallas_call(
        _tc_dense,
        out_shape=(
            jax.ShapeDtypeStruct((T, B, A), jnp.float32),
            jax.ShapeDtypeStruct((T, B), jnp.float32),
        ),
    )

    action_flat = sample()
    policy_logits, baseline = dense(theta)
    action = action_flat.reshape(T, B)
    return (policy_logits, baseline, action)
